# Initial kernel scaffold; baseline (speedup 1.0000x reference)
import jax
import jax.numpy as jnp
from jax.experimental import pallas as pl


def kernel(x, edge_index, W1a, b1a, g1, be1, W1b, b1b, W1c, b1c, W2a, b2a, g2, be2, W2b, b2b, W2c, b2c, W3a, b3a, g3, be3, W3b, b3b, W3c, b3c, W4, b4, Wr1, br1, Wr2, br2, Wr3, br3):
    raise NotImplementedError("write your pallas kernel here")


# speedup vs baseline: 2.6979x; 2.6979x over previous
import functools

import jax
import jax.numpy as jnp
from jax import lax
from jax.experimental import pallas as pl
from jax.experimental.pallas import tpu as pltpu
from jax.experimental.pallas import tpu_sc as plsc

N = 10000
E = 320000
H = 128
DOUT = 64

NC = 2
NS = 16
NW = NC * NS
K = 128
CHUNKS = (E + NW * K - 1) // (NW * K)
CHUNKS = 80
EPAD = NW * CHUNKS * K
NPAD = 10112
RPT = NPAD // NS

_PREC = lax.Precision.HIGHEST


@functools.cache
def _make_agg():
    mesh = plsc.VectorSubcoreMesh(core_axis_name="c", subcore_axis_name="s",
                                  num_cores=NC, num_subcores=NS)

    @functools.partial(
        pl.kernel,
        mesh=mesh,
        out_type=jax.ShapeDtypeStruct((NC, NPAD, H), jnp.float32),
        scratch_types=[
            pltpu.VMEM((CHUNKS, K), jnp.int32),
            pltpu.VMEM((CHUNKS, K), jnp.int32),
            pltpu.VMEM((K, H), jnp.float32),
            pltpu.VMEM_SHARED((NPAD, H), jnp.float32),
            pltpu.SemaphoreType.DMA,
        ],
    )
    def agg(h_hbm, src_hbm, dst_hbm, z_hbm, out_hbm, src_v, dst_v, rows_v,
            acc_sh, sem):
        c = lax.axis_index("c")
        s = lax.axis_index("s")
        g = c * NS + s
        pltpu.sync_copy(z_hbm, acc_sh.at[pl.ds(s * RPT, RPT)])
        pltpu.sync_copy(src_hbm.at[g], src_v)
        pltpu.sync_copy(dst_hbm.at[g], dst_v)
        plsc.subcore_barrier()

        def chunk(j, carry):
            pltpu.async_copy(h_hbm.at[src_v.at[j]], rows_v, sem).wait()
            pltpu.sync_copy(rows_v, acc_sh.at[dst_v.at[j]], add=True)
            return carry

        lax.fori_loop(0, CHUNKS, chunk, 0)
        plsc.subcore_barrier()
        pltpu.sync_copy(acc_sh.at[pl.ds(s * RPT, RPT)],
                        out_hbm.at[c, pl.ds(s * RPT, RPT)])

    return agg


def _agg_partials(h, srcr, dstr, zrows):
    out = _make_agg()(h, srcr, dstr, zrows)
    return out[0, :N], out[1, :N]


_GRID = 5
_R = N // _GRID


def _rows_spec():
    return pl.BlockSpec((_R, H), lambda i: (i, 0))


def _full_spec(shape):
    return pl.BlockSpec(shape, lambda i: tuple(0 for _ in shape))


def _bodyA(hp_ref, p0_ref, p1_ref, wr_ref, wa_ref, v_ref,
           t1_ref, id_ref, st_ref):
    i = pl.program_id(0)
    hp = hp_ref[...]
    agg = hp + p0_ref[...] + p1_ref[...]
    id_ref[...] = (jnp.dot(hp, wr_ref[...], preferred_element_type=jnp.float32,
                           precision=_PREC) + v_ref[5:6, :])
    t = (jnp.dot(agg, wa_ref[...], preferred_element_type=jnp.float32,
                 precision=_PREC) + v_ref[0:1, :])
    t1_ref[...] = t
    s0 = jnp.sum(t, axis=0, keepdims=True)
    s1 = jnp.sum(t * t, axis=0, keepdims=True)
    blk = jnp.concatenate([s0, s1, jnp.zeros((6, H), jnp.float32)], axis=0)

    @pl.when(i == 0)
    def _():
        st_ref[...] = blk

    @pl.when(i > 0)
    def _():
        st_ref[...] = st_ref[...] + blk


def _bodyB(t1_ref, id_ref, st_ref, wb_ref, wc_ref, v_ref, out_ref):
    t1 = t1_ref[...]
    m = st_ref[0:1, :] / N
    var = st_ref[1:2, :] / N - m * m
    t = (t1 - m) / jnp.sqrt(var + 1e-5) * v_ref[1:2, :] + v_ref[2:3, :]
    t = jnp.maximum(t, 0.0)
    t = (jnp.dot(t, wb_ref[...], preferred_element_type=jnp.float32,
                 precision=_PREC) + v_ref[3:4, :])
    t = jnp.maximum(t, 0.0)
    t = (jnp.dot(t, wc_ref[...], preferred_element_type=jnp.float32,
                 precision=_PREC) + v_ref[4:5, :])
    out_ref[...] = jnp.maximum(t + id_ref[...], 0.0)


def _bodyF(h_ref, p0_ref, p1_ref, w4_ref, b4_ref, out_ref):
    agg = h_ref[...] + p0_ref[...] + p1_ref[...]
    o = (jnp.dot(agg, w4_ref[...], preferred_element_type=jnp.float32,
                 precision=_PREC) + b4_ref[0:1, :])
    mx = jnp.max(o, axis=1, keepdims=True)
    o = o - mx
    lse = jnp.log(jnp.sum(jnp.exp(o), axis=1, keepdims=True))
    out_ref[...] = o - lse


def _layerA(hp, p0, p1, Wr, Wa, vecs):
    return pl.pallas_call(
        _bodyA,
        grid=(_GRID,),
        in_specs=[_rows_spec(), _rows_spec(), _rows_spec(),
                  _full_spec((H, H)), _full_spec((H, H)), _full_spec((8, H))],
        out_specs=[_rows_spec(), _rows_spec(), _full_spec((8, H))],
        out_shape=[jax.ShapeDtypeStruct((N, H), jnp.float32),
                   jax.ShapeDtypeStruct((N, H), jnp.float32),
                   jax.ShapeDtypeStruct((8, H), jnp.float32)],
    )(hp, p0, p1, Wr, Wa, vecs)


def _layerB(t1, ident, stats, Wb, Wc, vecs):
    return pl.pallas_call(
        _bodyB,
        grid=(_GRID,),
        in_specs=[_rows_spec(), _rows_spec(), _full_spec((8, H)),
                  _full_spec((H, H)), _full_spec((H, H)), _full_spec((8, H))],
        out_specs=_rows_spec(),
        out_shape=jax.ShapeDtypeStruct((N, H), jnp.float32),
    )(t1, ident, stats, Wb, Wc, vecs)


def _layerF(h, p0, p1, W4, b4row):
    return pl.pallas_call(
        _bodyF,
        grid=(_GRID,),
        in_specs=[_rows_spec(), _rows_spec(), _rows_spec(),
                  _full_spec((H, DOUT)), _full_spec((8, DOUT))],
        out_specs=pl.BlockSpec((_R, DOUT), lambda i: (i, 0)),
        out_shape=jax.ShapeDtypeStruct((N, DOUT), jnp.float32),
    )(h, p0, p1, W4, b4row)


def kernel(x, edge_index,
           W1a, b1a, g1, be1, W1b, b1b, W1c, b1c,
           W2a, b2a, g2, be2, W2b, b2b, W2c, b2c,
           W3a, b3a, g3, be3, W3b, b3b, W3c, b3c,
           W4, b4, Wr1, br1, Wr2, br2, Wr3, br3):
    src = edge_index[0]
    dst = edge_index[1]
    npad = EPAD - E
    pad_src = jnp.zeros((npad,), jnp.int32)
    pad_dst = N + (jnp.arange(npad, dtype=jnp.int32) % (NPAD - N))
    srcr = jnp.concatenate([src, pad_src]).reshape(NW, CHUNKS, K)
    dstr = jnp.concatenate([dst, pad_dst]).reshape(NW, CHUNKS, K)
    zrows = jnp.zeros((RPT, H), jnp.float32)

    def vecstack(ba, g, be, bb, bc, br):
        return jnp.stack([ba, g, be, bb, bc, br,
                          jnp.zeros((H,), jnp.float32),
                          jnp.zeros((H,), jnp.float32)])

    v1 = vecstack(b1a, g1, be1, b1b, b1c, br1)
    v2 = vecstack(b2a, g2, be2, b2b, b2c, br2)
    v3 = vecstack(b3a, g3, be3, b3b, b3c, br3)
    b4row = jnp.concatenate(
        [b4.reshape(1, DOUT), jnp.zeros((7, DOUT), jnp.float32)])

    h = x
    for (Wa, Wb, Wc, Wr, vecs) in ((W1a, W1b, W1c, Wr1, v1),
                                   (W2a, W2b, W2c, Wr2, v2),
                                   (W3a, W3b, W3c, Wr3, v3)):
        p0, p1 = _agg_partials(h, srcr, dstr, zrows)
        t1, ident, stats = _layerA(h, p0, p1, Wr, Wa, vecs)
        h = _layerB(t1, ident, stats, Wb, Wc, vecs)
    p0, p1 = _agg_partials(h, srcr, dstr, zrows)
    return _layerF(h, p0, p1, W4, b4row)


# speedup vs baseline: 2.9080x; 1.0779x over previous
import functools

import jax
import jax.numpy as jnp
from jax import lax
from jax.experimental import pallas as pl
from jax.experimental.pallas import tpu as pltpu
from jax.experimental.pallas import tpu_sc as plsc

N = 10000
E = 320000
H = 128
DOUT = 64

NC = 2
NS = 16
NW = NC * NS
K = 128
CHUNKS = (E + NW * K - 1) // (NW * K)
CHUNKS = 80
EPAD = NW * CHUNKS * K
NPAD = 10112
RPT = NPAD // NS

_PREC = lax.Precision.HIGHEST


@functools.cache
def _make_agg():
    mesh = plsc.VectorSubcoreMesh(core_axis_name="c", subcore_axis_name="s",
                                  num_cores=NC, num_subcores=NS)

    @functools.partial(
        pl.kernel,
        mesh=mesh,
        out_type=jax.ShapeDtypeStruct((NC, NPAD, H), jnp.float32),
        scratch_types=[
            pltpu.VMEM((CHUNKS // 2, K), jnp.int32),
            pltpu.VMEM((CHUNKS // 2, K), jnp.int32),
            pltpu.VMEM((K, H), jnp.float32),
            pltpu.VMEM((K, H), jnp.float32),
            pltpu.VMEM_SHARED((NPAD, H), jnp.float32),
            pltpu.SemaphoreType.DMA,
        ],
    )
    def agg(h_hbm, src_hbm, dst_hbm, z_hbm, out_hbm, src_v, dst_v, rows0,
            rows1, acc_sh, semg):
        c = lax.axis_index("c")
        s = lax.axis_index("s")
        g = c * NS + s
        hc = CHUNKS // 2
        pltpu.sync_copy(z_hbm, acc_sh.at[pl.ds(s * RPT, RPT)])
        plsc.subcore_barrier()

        for half in range(2):
            pltpu.sync_copy(src_hbm.at[g, pl.ds(half * hc, hc)], src_v)
            pltpu.sync_copy(dst_hbm.at[g, pl.ds(half * hc, hc)], dst_v)
            pltpu.async_copy(h_hbm.at[src_v.at[0]], rows0, semg)

            def step(i, carry):
                j0 = 2 * i
                pltpu.make_async_copy(
                    h_hbm.at[src_v.at[j0]], rows0, semg).wait()
                pltpu.async_copy(h_hbm.at[src_v.at[j0 + 1]], rows1, semg)
                pltpu.sync_copy(rows0, acc_sh.at[dst_v.at[j0]], add=True)
                pltpu.make_async_copy(
                    h_hbm.at[src_v.at[j0]], rows1, semg).wait()

                @pl.when(i < hc // 2 - 1)
                def _():
                    pltpu.async_copy(h_hbm.at[src_v.at[j0 + 2]], rows0, semg)

                pltpu.sync_copy(rows1, acc_sh.at[dst_v.at[j0 + 1]], add=True)
                return carry

            lax.fori_loop(0, hc // 2, step, 0)
        plsc.subcore_barrier()
        pltpu.sync_copy(acc_sh.at[pl.ds(s * RPT, RPT)],
                        out_hbm.at[c, pl.ds(s * RPT, RPT)])

    return agg


def _agg_partials(h, srcr, dstr, zrows):
    out = _make_agg()(h, srcr, dstr, zrows)
    return out[0, :N], out[1, :N]


_GRID = 5
_R = N // _GRID


def _rows_spec():
    return pl.BlockSpec((_R, H), lambda i: (i, 0))


def _full_spec(shape):
    return pl.BlockSpec(shape, lambda i: tuple(0 for _ in shape))


def _bodyA(hp_ref, p0_ref, p1_ref, wr_ref, wa_ref, v_ref,
           t1_ref, id_ref, st_ref):
    i = pl.program_id(0)
    hp = hp_ref[...]
    agg = hp + p0_ref[...] + p1_ref[...]
    id_ref[...] = (jnp.dot(hp, wr_ref[...], preferred_element_type=jnp.float32,
                           precision=_PREC) + v_ref[5:6, :])
    t = (jnp.dot(agg, wa_ref[...], preferred_element_type=jnp.float32,
                 precision=_PREC) + v_ref[0:1, :])
    t1_ref[...] = t
    s0 = jnp.sum(t, axis=0, keepdims=True)
    s1 = jnp.sum(t * t, axis=0, keepdims=True)
    blk = jnp.concatenate([s0, s1, jnp.zeros((6, H), jnp.float32)], axis=0)

    @pl.when(i == 0)
    def _():
        st_ref[...] = blk

    @pl.when(i > 0)
    def _():
        st_ref[...] = st_ref[...] + blk


def _bodyB(t1_ref, id_ref, st_ref, wb_ref, wc_ref, v_ref, out_ref):
    t1 = t1_ref[...]
    m = st_ref[0:1, :] / N
    var = st_ref[1:2, :] / N - m * m
    t = (t1 - m) / jnp.sqrt(var + 1e-5) * v_ref[1:2, :] + v_ref[2:3, :]
    t = jnp.maximum(t, 0.0)
    t = (jnp.dot(t, wb_ref[...], preferred_element_type=jnp.float32,
                 precision=_PREC) + v_ref[3:4, :])
    t = jnp.maximum(t, 0.0)
    t = (jnp.dot(t, wc_ref[...], preferred_element_type=jnp.float32,
                 precision=_PREC) + v_ref[4:5, :])
    out_ref[...] = jnp.maximum(t + id_ref[...], 0.0)


def _bodyF(h_ref, p0_ref, p1_ref, w4_ref, b4_ref, out_ref):
    agg = h_ref[...] + p0_ref[...] + p1_ref[...]
    o = (jnp.dot(agg, w4_ref[...], preferred_element_type=jnp.float32,
                 precision=_PREC) + b4_ref[0:1, :])
    mx = jnp.max(o, axis=1, keepdims=True)
    o = o - mx
    lse = jnp.log(jnp.sum(jnp.exp(o), axis=1, keepdims=True))
    out_ref[...] = o - lse


def _layerA(hp, p0, p1, Wr, Wa, vecs):
    return pl.pallas_call(
        _bodyA,
        grid=(_GRID,),
        in_specs=[_rows_spec(), _rows_spec(), _rows_spec(),
                  _full_spec((H, H)), _full_spec((H, H)), _full_spec((8, H))],
        out_specs=[_rows_spec(), _rows_spec(), _full_spec((8, H))],
        out_shape=[jax.ShapeDtypeStruct((N, H), jnp.float32),
                   jax.ShapeDtypeStruct((N, H), jnp.float32),
                   jax.ShapeDtypeStruct((8, H), jnp.float32)],
    )(hp, p0, p1, Wr, Wa, vecs)


def _layerB(t1, ident, stats, Wb, Wc, vecs):
    return pl.pallas_call(
        _bodyB,
        grid=(_GRID,),
        in_specs=[_rows_spec(), _rows_spec(), _full_spec((8, H)),
                  _full_spec((H, H)), _full_spec((H, H)), _full_spec((8, H))],
        out_specs=_rows_spec(),
        out_shape=jax.ShapeDtypeStruct((N, H), jnp.float32),
    )(t1, ident, stats, Wb, Wc, vecs)


def _layerF(h, p0, p1, W4, b4row):
    return pl.pallas_call(
        _bodyF,
        grid=(_GRID,),
        in_specs=[_rows_spec(), _rows_spec(), _rows_spec(),
                  _full_spec((H, DOUT)), _full_spec((8, DOUT))],
        out_specs=pl.BlockSpec((_R, DOUT), lambda i: (i, 0)),
        out_shape=jax.ShapeDtypeStruct((N, DOUT), jnp.float32),
    )(h, p0, p1, W4, b4row)


def kernel(x, edge_index,
           W1a, b1a, g1, be1, W1b, b1b, W1c, b1c,
           W2a, b2a, g2, be2, W2b, b2b, W2c, b2c,
           W3a, b3a, g3, be3, W3b, b3b, W3c, b3c,
           W4, b4, Wr1, br1, Wr2, br2, Wr3, br3):
    src = edge_index[0]
    dst = edge_index[1]
    npad = EPAD - E
    pad_src = jnp.zeros((npad,), jnp.int32)
    pad_dst = N + (jnp.arange(npad, dtype=jnp.int32) % (NPAD - N))
    srcr = jnp.concatenate([src, pad_src]).reshape(NW, CHUNKS, K)
    dstr = jnp.concatenate([dst, pad_dst]).reshape(NW, CHUNKS, K)
    zrows = jnp.zeros((RPT, H), jnp.float32)

    def vecstack(ba, g, be, bb, bc, br):
        return jnp.stack([ba, g, be, bb, bc, br,
                          jnp.zeros((H,), jnp.float32),
                          jnp.zeros((H,), jnp.float32)])

    v1 = vecstack(b1a, g1, be1, b1b, b1c, br1)
    v2 = vecstack(b2a, g2, be2, b2b, b2c, br2)
    v3 = vecstack(b3a, g3, be3, b3b, b3c, br3)
    b4row = jnp.concatenate(
        [b4.reshape(1, DOUT), jnp.zeros((7, DOUT), jnp.float32)])

    h = x
    for (Wa, Wb, Wc, Wr, vecs) in ((W1a, W1b, W1c, Wr1, v1),
                                   (W2a, W2b, W2c, Wr2, v2),
                                   (W3a, W3b, W3c, Wr3, v3)):
        p0, p1 = _agg_partials(h, srcr, dstr, zrows)
        t1, ident, stats = _layerA(h, p0, p1, Wr, Wa, vecs)
        h = _layerB(t1, ident, stats, Wb, Wc, vecs)
    p0, p1 = _agg_partials(h, srcr, dstr, zrows)
    return _layerF(h, p0, p1, W4, b4row)
